# baseline (speedup 1.0000x reference)
import functools

import jax
import jax.numpy as jnp
from jax import lax
from jax.experimental import pallas as pl
from jax.experimental.pallas import tpu as pltpu
from jax.experimental.pallas import tpu_sc as plsc

NN = 100000
EE = 1600000
GG = 256
NP = 100352
STRIPE = 6272
ZCH = 392
NZC = STRIPE // ZCH
ROWS_PT = 392
EP = 32 * ROWS_PT * 128
KJ = 8
NSC = ROWS_PT // KJ
BN = 10000
NB = NN // BN
EPS = 1e-5

_mesh = plsc.VectorSubcoreMesh(core_axis_name="c", subcore_axis_name="s")
_sc_params = pltpu.CompilerParams(use_tc_tiling_on_sc=False)


def _zero_slab(zbuf, slab, s):
    for k in range(NZC):
        pltpu.sync_copy(zbuf, slab.at[pl.ds(s * STRIPE + k * ZCH, ZCH)])


def _flush_slab(slab, out_ref, s):
    for k in range(NZC):
        sl = pl.ds(s * STRIPE + k * ZCH, ZCH)
        pltpu.sync_copy(slab.at[sl], out_ref.at[sl])


@functools.partial(
    pl.kernel, mesh=_mesh, compiler_params=_sc_params,
    out_type=jax.ShapeDtypeStruct((2, NP, 16), jnp.float32),
    scratch_types=[
        pltpu.VMEM((KJ, 128), jnp.int32),
        pltpu.VMEM((128, 16), jnp.float32),
        pltpu.VMEM((ZCH, 16), jnp.float32),
        pltpu.VMEM_SHARED((NP, 16), jnp.float32),
    ],
)
def _sc_degree(dst2d, ones_in, zeros_in, out, dbuf, obuf, zbuf, slab):
    c = lax.axis_index("c")
    s = lax.axis_index("s")
    w = c * 16 + s
    pltpu.sync_copy(zeros_in, zbuf)
    pltpu.sync_copy(ones_in, obuf)
    _zero_slab(zbuf, slab, s)
    plsc.subcore_barrier()

    def chunk(i, carry):
        r0 = w * ROWS_PT + i * KJ
        pltpu.sync_copy(dst2d.at[pl.ds(r0, KJ)], dbuf)
        for j in range(KJ):
            pltpu.sync_copy(obuf, slab.at[dbuf.at[j]], add=True)
        return carry

    lax.fori_loop(0, NSC, chunk, 0)
    plsc.subcore_barrier()
    _flush_slab(slab, out.at[c], s)


def _make_sc_agg(npass):
    @functools.partial(
        pl.kernel, mesh=_mesh, compiler_params=_sc_params,
        out_type=jax.ShapeDtypeStruct((npass, 2, NP, 16), jnp.float32),
        scratch_types=[
            pltpu.VMEM((KJ, 128), jnp.int32),
            pltpu.VMEM((KJ, 128), jnp.int32),
            pltpu.VMEM((KJ, 128, 16), jnp.float32),
            pltpu.VMEM((ZCH, 16), jnp.float32),
            pltpu.VMEM_SHARED((NP, 16), jnp.float32),
            pltpu.SemaphoreType.DMA,
        ],
    )
    def agg(*refs):
        zrefs = refs[:npass]
        src2d, dst2d, zeros_in, out = refs[npass:npass + 4]
        sbuf, dbuf, rows, zbuf, slab, sem = refs[npass + 4:]
        c = lax.axis_index("c")
        s = lax.axis_index("s")
        w = c * 16 + s
        pltpu.sync_copy(zeros_in, zbuf)
        for p in range(npass):
            _zero_slab(zbuf, slab, s)
            plsc.subcore_barrier()

            def chunk(i, carry):
                r0 = w * ROWS_PT + i * KJ
                pltpu.sync_copy(src2d.at[pl.ds(r0, KJ)], sbuf)
                pltpu.sync_copy(dst2d.at[pl.ds(r0, KJ)], dbuf)
                handles = [
                    pltpu.async_copy(zrefs[p].at[sbuf.at[j]], rows.at[j], sem)
                    for j in range(KJ)
                ]
                for h in handles:
                    h.wait()
                for j in range(KJ):
                    pltpu.sync_copy(rows.at[j], slab.at[dbuf.at[j]], add=True)
                return carry

            lax.fori_loop(0, NSC, chunk, 0)
            plsc.subcore_barrier()
            _flush_slab(slab, out.at[p, c], s)
            if p + 1 < npass:
                plsc.subcore_barrier()

    return agg


_sc_agg1 = _make_sc_agg(1)
_sc_agg8 = _make_sc_agg(8)


def _astats_body(q0, q1, xp, dinv, w1p, b1, g1, be1, out, csum, gram):
    i = pl.program_id(0)

    @pl.when(i == 0)
    def _init():
        csum[...] = jnp.zeros_like(csum)
        gram[...] = jnp.zeros_like(gram)

    a = dinv[...] * (q0[...] + q1[...] + xp[...])
    csum[...] += jnp.sum(a, axis=0, keepdims=True)
    gram[...] += lax.dot_general(a, a, (((0,), (0,)), ((), ())),
                                 preferred_element_type=jnp.float32,
                                 precision=lax.Precision.HIGHEST)

    @pl.when(i == NB - 1)
    def _fin():
        n = jnp.float32(NN)
        cw = jnp.dot(csum[...], w1p[...],
                     preferred_element_type=jnp.float32,
                     precision=lax.Precision.HIGHEST)
        mean = cw / n + b1[...]
        t = jnp.dot(gram[...], w1p[...],
                    preferred_element_type=jnp.float32,
                    precision=lax.Precision.HIGHEST)
        d = jnp.sum(w1p[...] * t, axis=0, keepdims=True)
        e2 = d / n + 2.0 * b1[...] * cw / n + b1[...] * b1[...]
        var = e2 - mean * mean
        scale = g1[...] * lax.rsqrt(var + EPS)
        shift = be1[...] - mean * scale
        out[0:1, :] = scale
        out[1:2, :] = shift


def _aapply_body(q0, q1, xp, dinv, w1p, b1, ss, w2, out):
    a = dinv[...] * (q0[...] + q1[...] + xp[...])
    hp = jnp.dot(a, w1p[...], preferred_element_type=jnp.float32) + b1[...]
    h = jnp.maximum(hp * ss[0:1, :] + ss[1:2, :], 0.0)
    out[...] = jnp.dot(h, w2[...], preferred_element_type=jnp.float32) \
        * dinv[...]


def _final_body(s2, zp, dinv, bf, b2, g2, be2, out,
                csum, csum2, pool, cnt):
    i = pl.program_id(0)

    @pl.when(i == 0)
    def _init():
        csum[...] = jnp.zeros_like(csum)
        csum2[...] = jnp.zeros_like(csum2)
        pool[...] = jnp.zeros_like(pool)
        cnt[...] = jnp.zeros_like(cnt)

    y = dinv[...] * (s2[...] + zp[...]) + b2[...]
    csum[...] += jnp.sum(y, axis=0, keepdims=True)
    csum2[...] += jnp.sum(y * y, axis=0, keepdims=True)
    gids = lax.broadcasted_iota(jnp.int32, (BN, GG), 1)
    oh = jnp.where(bf[...] == gids, 1.0, 0.0)
    pool[...] += lax.dot_general(oh, y, (((0,), (0,)), ((), ())),
                                 preferred_element_type=jnp.float32)
    cnt[...] += lax.dot_general(oh, jnp.ones((BN, 1), jnp.float32),
                                (((0,), (0,)), ((), ())),
                                preferred_element_type=jnp.float32,
                                precision=lax.Precision.HIGHEST)

    @pl.when(i == NB - 1)
    def _fin():
        n = jnp.float32(NN)
        mean = csum[...] / n
        var = csum2[...] / n - mean * mean
        scale = g2[...] * lax.rsqrt(var + EPS)
        shift = be2[...] - mean * scale
        pm = pool[...] / jnp.maximum(cnt[...], 1.0)
        pb = pm * scale + shift
        nrm = jnp.sqrt(jnp.sum(pb * pb, axis=1, keepdims=True))
        out[...] = pb / jnp.maximum(nrm, 1e-12)


def kernel(x, edge_index, batch, W1, b1, g1, be1, W2, b2, g2, be2):
    f32 = jnp.float32
    pad_e = EP - EE
    srcp = jnp.concatenate(
        [edge_index[0], jnp.full((pad_e,), NN, jnp.int32)]).reshape(-1, 128)
    dstp = jnp.concatenate(
        [edge_index[1], jnp.full((pad_e,), NN, jnp.int32)]).reshape(-1, 128)
    ones_in = jnp.ones((128, 16), f32)
    zeros_in = jnp.zeros((ZCH, 16), f32)

    degp = _sc_degree(dstp, ones_in, zeros_in)
    deg = degp[0, :NN, 0] + degp[1, :NN, 0] + 1.0
    dinv = lax.rsqrt(deg)
    dcol = dinv.reshape(NN, 1)

    xprime = jnp.zeros((NP, 16), f32).at[:NN, :9].set(x * dcol)
    q1 = _sc_agg1(xprime, srcp, dstp, zeros_in)[0]
    q10, q11 = q1[0, :NN], q1[1, :NN]
    xpn = xprime[:NN]

    w1p = jnp.zeros((16, 256), f32).at[:9].set(W1)
    row = lambda v: v.reshape(1, -1)
    nspec = lambda wdt: pl.BlockSpec((BN, wdt), lambda i: (i, 0))
    full = lambda a, b: pl.BlockSpec((a, b), lambda i: (0, 0))

    ss = pl.pallas_call(
        _astats_body,
        grid=(NB,),
        in_specs=[nspec(16), nspec(16), nspec(16), nspec(1),
                  full(16, 256), full(1, 256), full(1, 256), full(1, 256)],
        out_specs=full(2, 256),
        out_shape=jax.ShapeDtypeStruct((2, 256), f32),
        scratch_shapes=[pltpu.VMEM((1, 16), f32), pltpu.VMEM((16, 16), f32)],
    )(q10, q11, xpn, dcol, w1p, row(b1), row(g1), row(be1))

    zprime = pl.pallas_call(
        _aapply_body,
        grid=(NB,),
        in_specs=[nspec(16), nspec(16), nspec(16), nspec(1),
                  full(16, 256), full(1, 256), full(2, 256), full(256, 128)],
        out_specs=nspec(128),
        out_shape=jax.ShapeDtypeStruct((NN, 128), f32),
    )(q10, q11, xpn, dcol, w1p, row(b1), ss, W2)

    zpad = jnp.zeros((NP, 128), f32).at[:NN].set(zprime)
    zparts = [zpad[:, 16 * p:16 * (p + 1)] for p in range(8)]
    q2 = _sc_agg8(*zparts, srcp, dstp, zeros_in)
    q2s = q2[:, 0] + q2[:, 1]
    s2 = jnp.transpose(q2s, (1, 0, 2)).reshape(NP, 128)[:NN]

    bf = batch.reshape(NN, 1)
    return pl.pallas_call(
        _final_body,
        grid=(NB,),
        in_specs=[nspec(128), nspec(128), nspec(1), nspec(1),
                  full(1, 128), full(1, 128), full(1, 128)],
        out_specs=full(GG, 128),
        out_shape=jax.ShapeDtypeStruct((GG, 128), f32),
        scratch_shapes=[pltpu.VMEM((1, 128), f32), pltpu.VMEM((1, 128), f32),
                        pltpu.VMEM((GG, 128), f32), pltpu.VMEM((GG, 1), f32)],
    )(s2, zprime, dcol, bf, row(b2), row(g2), row(be2))


# speedup vs baseline: 1.1615x; 1.1615x over previous
import functools

import jax
import jax.numpy as jnp
from jax import lax
from jax.experimental import pallas as pl
from jax.experimental.pallas import tpu as pltpu
from jax.experimental.pallas import tpu_sc as plsc

NN = 100000
EE = 1600000
GG = 256
NP = 100352
STRIPE = 6272
ZCH = 392
NZC = STRIPE // ZCH
ROWS_PT = 392
EP = 32 * ROWS_PT * 128
KJ = 8
NSC = ROWS_PT // KJ
BN = 10000
NB = NN // BN
BNA = 2000
NBA = NN // BNA
BNF = 1000
NBF = NN // BNF
EPS = 1e-5

_mesh = plsc.VectorSubcoreMesh(core_axis_name="c", subcore_axis_name="s")
_sc_params = pltpu.CompilerParams(use_tc_tiling_on_sc=False)


def _zero_slab(zbuf, slab, s):
    for k in range(NZC):
        pltpu.sync_copy(zbuf, slab.at[pl.ds(s * STRIPE + k * ZCH, ZCH)])


def _flush_slab(slab, out_ref, s):
    for k in range(NZC):
        sl = pl.ds(s * STRIPE + k * ZCH, ZCH)
        pltpu.sync_copy(slab.at[sl], out_ref.at[sl])


@functools.partial(
    pl.kernel, mesh=_mesh, compiler_params=_sc_params,
    out_type=jax.ShapeDtypeStruct((2, NP, 16), jnp.float32),
    scratch_types=[
        pltpu.VMEM((KJ, 128), jnp.int32),
        pltpu.VMEM((128, 16), jnp.float32),
        pltpu.VMEM((ZCH, 16), jnp.float32),
        pltpu.VMEM_SHARED((NP, 16), jnp.float32),
    ],
)
def _sc_degree(dst2d, ones_in, zeros_in, out, dbuf, obuf, zbuf, slab):
    c = lax.axis_index("c")
    s = lax.axis_index("s")
    w = c * 16 + s
    pltpu.sync_copy(zeros_in, zbuf)
    pltpu.sync_copy(ones_in, obuf)
    _zero_slab(zbuf, slab, s)
    plsc.subcore_barrier()

    def chunk(i, carry):
        r0 = w * ROWS_PT + i * KJ
        pltpu.sync_copy(dst2d.at[pl.ds(r0, KJ)], dbuf)
        for j in range(KJ):
            pltpu.sync_copy(obuf, slab.at[dbuf.at[j]], add=True)
        return carry

    lax.fori_loop(0, NSC, chunk, 0)
    plsc.subcore_barrier()
    _flush_slab(slab, out.at[c], s)


def _make_sc_agg(npass):
    @functools.partial(
        pl.kernel, mesh=_mesh, compiler_params=_sc_params,
        out_type=jax.ShapeDtypeStruct((npass, 2, NP, 16), jnp.float32),
        scratch_types=[
            pltpu.VMEM((KJ, 128), jnp.int32),
            pltpu.VMEM((KJ, 128), jnp.int32),
            pltpu.VMEM((KJ, 128, 16), jnp.float32),
            pltpu.VMEM((ZCH, 16), jnp.float32),
            pltpu.VMEM_SHARED((NP, 16), jnp.float32),
            pltpu.SemaphoreType.DMA,
        ],
    )
    def agg(*refs):
        zrefs = refs[:npass]
        src2d, dst2d, zeros_in, out = refs[npass:npass + 4]
        sbuf, dbuf, rows, zbuf, slab, sem = refs[npass + 4:]
        c = lax.axis_index("c")
        s = lax.axis_index("s")
        w = c * 16 + s
        pltpu.sync_copy(zeros_in, zbuf)
        for p in range(npass):
            _zero_slab(zbuf, slab, s)
            plsc.subcore_barrier()

            def chunk(i, carry):
                r0 = w * ROWS_PT + i * KJ
                pltpu.sync_copy(src2d.at[pl.ds(r0, KJ)], sbuf)
                pltpu.sync_copy(dst2d.at[pl.ds(r0, KJ)], dbuf)
                handles = [
                    pltpu.async_copy(zrefs[p].at[sbuf.at[j]], rows.at[j], sem)
                    for j in range(KJ)
                ]
                for h in handles:
                    h.wait()
                for j in range(KJ):
                    pltpu.sync_copy(rows.at[j], slab.at[dbuf.at[j]], add=True)
                return carry

            lax.fori_loop(0, NSC, chunk, 0)
            plsc.subcore_barrier()
            _flush_slab(slab, out.at[p, c], s)
            if p + 1 < npass:
                plsc.subcore_barrier()

    return agg


_sc_agg1 = _make_sc_agg(1)
_sc_agg8 = _make_sc_agg(8)


def _astats_body(q0, q1, xp, dinv, w1p, b1, g1, be1, out, csum, gram):
    i = pl.program_id(0)

    @pl.when(i == 0)
    def _init():
        csum[...] = jnp.zeros_like(csum)
        gram[...] = jnp.zeros_like(gram)

    a = dinv[...] * (q0[...] + q1[...] + xp[...])
    csum[...] += jnp.sum(a, axis=0, keepdims=True)
    gram[...] += lax.dot_general(a, a, (((0,), (0,)), ((), ())),
                                 preferred_element_type=jnp.float32,
                                 precision=lax.Precision.HIGHEST)

    @pl.when(i == NB - 1)
    def _fin():
        n = jnp.float32(NN)
        cw = jnp.dot(csum[...], w1p[...],
                     preferred_element_type=jnp.float32,
                     precision=lax.Precision.HIGHEST)
        mean = cw / n + b1[...]
        t = jnp.dot(gram[...], w1p[...],
                    preferred_element_type=jnp.float32,
                    precision=lax.Precision.HIGHEST)
        d = jnp.sum(w1p[...] * t, axis=0, keepdims=True)
        e2 = d / n + 2.0 * b1[...] * cw / n + b1[...] * b1[...]
        var = e2 - mean * mean
        scale = g1[...] * lax.rsqrt(var + EPS)
        shift = be1[...] - mean * scale
        out[0:1, :] = scale
        out[1:2, :] = shift


def _aapply_body(q0, q1, xp, dinv, w1p, b1, ss, w2, *outs):
    a = dinv[...] * (q0[...] + q1[...] + xp[...])
    hp = jnp.dot(a, w1p[...], preferred_element_type=jnp.float32) + b1[...]
    h = jnp.maximum(hp * ss[0:1, :] + ss[1:2, :], 0.0)
    z = jnp.dot(h, w2[...], preferred_element_type=jnp.float32) * dinv[...]
    for p in range(8):
        outs[p][...] = z[:, 16 * p:16 * (p + 1)]


def _final_body(*refs):
    q2 = refs[0]
    zs = refs[1:9]
    dinv, bf, b2, g2, be2, out, csum, csum2, pool, cnt = refs[9:]
    i = pl.program_id(0)

    @pl.when(i == 0)
    def _init():
        csum[...] = jnp.zeros_like(csum)
        csum2[...] = jnp.zeros_like(csum2)
        pool[...] = jnp.zeros_like(pool)
        cnt[...] = jnp.zeros_like(cnt)

    q = q2[...]
    cols = [q[p, 0] + q[p, 1] + zs[p][...] for p in range(8)]
    y = dinv[...] * jnp.concatenate(cols, axis=1) + b2[...]
    csum[...] += jnp.sum(y, axis=0, keepdims=True)
    csum2[...] += jnp.sum(y * y, axis=0, keepdims=True)
    gids = lax.broadcasted_iota(jnp.int32, (BNF, GG), 1)
    oh = jnp.where(bf[...] == gids, 1.0, 0.0)
    pool[...] += lax.dot_general(oh, y, (((0,), (0,)), ((), ())),
                                 preferred_element_type=jnp.float32)
    cnt[...] += lax.dot_general(oh, jnp.ones((BNF, 1), jnp.float32),
                                (((0,), (0,)), ((), ())),
                                preferred_element_type=jnp.float32,
                                precision=lax.Precision.HIGHEST)

    @pl.when(i == NBF - 1)
    def _fin():
        n = jnp.float32(NN)
        mean = csum[...] / n
        var = csum2[...] / n - mean * mean
        scale = g2[...] * lax.rsqrt(var + EPS)
        shift = be2[...] - mean * scale
        pm = pool[...] / jnp.maximum(cnt[...], 1.0)
        pb = pm * scale + shift
        nrm = jnp.sqrt(jnp.sum(pb * pb, axis=1, keepdims=True))
        out[...] = pb / jnp.maximum(nrm, 1e-12)


def kernel(x, edge_index, batch, W1, b1, g1, be1, W2, b2, g2, be2):
    f32 = jnp.float32
    pad_e = EP - EE
    srcp = jnp.concatenate(
        [edge_index[0], jnp.full((pad_e,), NN, jnp.int32)]).reshape(-1, 128)
    dstp = jnp.concatenate(
        [edge_index[1], jnp.full((pad_e,), NN, jnp.int32)]).reshape(-1, 128)
    ones_in = jnp.ones((128, 16), f32)
    zeros_in = jnp.zeros((ZCH, 16), f32)

    degp = _sc_degree(dstp, ones_in, zeros_in)
    deg = degp[0, :NN, 0] + degp[1, :NN, 0] + 1.0
    dinv = lax.rsqrt(deg)
    dcol = dinv.reshape(NN, 1)

    xprime = jnp.zeros((NP, 16), f32).at[:NN, :9].set(x * dcol)
    q1 = _sc_agg1(xprime, srcp, dstp, zeros_in)[0]
    q10, q11 = q1[0, :NN], q1[1, :NN]
    xpn = xprime[:NN]

    w1p = jnp.zeros((16, 256), f32).at[:9].set(W1)
    row = lambda v: v.reshape(1, -1)
    nspec = lambda wdt: pl.BlockSpec((BN, wdt), lambda i: (i, 0))
    full = lambda a, b: pl.BlockSpec((a, b), lambda i: (0, 0))

    ss = pl.pallas_call(
        _astats_body,
        grid=(NB,),
        in_specs=[nspec(16), nspec(16), nspec(16), nspec(1),
                  full(16, 256), full(1, 256), full(1, 256), full(1, 256)],
        out_specs=full(2, 256),
        out_shape=jax.ShapeDtypeStruct((2, 256), f32),
        scratch_shapes=[pltpu.VMEM((1, 16), f32), pltpu.VMEM((16, 16), f32)],
    )(q10, q11, xpn, dcol, w1p, row(b1), row(g1), row(be1))

    aspec = lambda wdt: pl.BlockSpec((BNA, wdt), lambda i: (i, 0))
    zparts = pl.pallas_call(
        _aapply_body,
        grid=(NBA,),
        in_specs=[aspec(16), aspec(16), aspec(16), aspec(1),
                  full(16, 256), full(1, 256), full(2, 256), full(256, 128)],
        out_specs=[aspec(16)] * 8,
        out_shape=[jax.ShapeDtypeStruct((NP, 16), f32)] * 8,
    )(q10, q11, xpn, dcol, w1p, row(b1), ss, W2)

    q2 = _sc_agg8(*zparts, srcp, dstp, zeros_in)

    bf = batch.reshape(NN, 1)
    fspec = lambda wdt: pl.BlockSpec((BNF, wdt), lambda i: (i, 0))
    return pl.pallas_call(
        _final_body,
        grid=(NBF,),
        in_specs=[pl.BlockSpec((8, 2, BNF, 16), lambda i: (0, 0, i, 0))] +
                 [fspec(16)] * 8 +
                 [fspec(1), fspec(1),
                  full(1, 128), full(1, 128), full(1, 128)],
        out_specs=full(GG, 128),
        out_shape=jax.ShapeDtypeStruct((GG, 128), f32),
        scratch_shapes=[pltpu.VMEM((1, 128), f32), pltpu.VMEM((1, 128), f32),
                        pltpu.VMEM((GG, 128), f32), pltpu.VMEM((GG, 1), f32)],
    )(q2, *zparts, dcol, bf, row(b2), row(g2), row(be2))


# speedup vs baseline: 1.2931x; 1.1133x over previous
import functools

import jax
import jax.numpy as jnp
from jax import lax
from jax.experimental import pallas as pl
from jax.experimental.pallas import tpu as pltpu
from jax.experimental.pallas import tpu_sc as plsc

NN = 100000
EE = 1600000
GG = 256
NP = 100352
STRIPE = 6272
ZCH = 392
NZC = STRIPE // ZCH
ROWS_PT = 392
EP = 32 * ROWS_PT * 128
KJ = 8
NSC = ROWS_PT // KJ
BN = 10000
NB = NN // BN
BNA = 2000
NBA = NN // BNA
BNF = 1000
NBF = NN // BNF
EPS = 1e-5

_mesh = plsc.VectorSubcoreMesh(core_axis_name="c", subcore_axis_name="s")
_sc_params = pltpu.CompilerParams(use_tc_tiling_on_sc=False)


def _zero_slab(zbuf, slab, s):
    for k in range(NZC):
        pltpu.sync_copy(zbuf, slab.at[pl.ds(s * STRIPE + k * ZCH, ZCH)])


def _flush_slab(slab, out_ref, s):
    for k in range(NZC):
        sl = pl.ds(s * STRIPE + k * ZCH, ZCH)
        pltpu.sync_copy(slab.at[sl], out_ref.at[sl])


@functools.partial(
    pl.kernel, mesh=_mesh, compiler_params=_sc_params,
    out_type=jax.ShapeDtypeStruct((2, NP, 16), jnp.float32),
    scratch_types=[
        pltpu.VMEM((KJ, 128), jnp.int32),
        pltpu.VMEM((128, 16), jnp.float32),
        pltpu.VMEM((ZCH, 16), jnp.float32),
        pltpu.VMEM_SHARED((NP, 16), jnp.float32),
        pltpu.SemaphoreType.DMA,
    ],
)
def _sc_degree(dst2d, ones_in, zeros_in, out, dbuf, obuf, zbuf, slab, sem2):
    c = lax.axis_index("c")
    s = lax.axis_index("s")
    w = c * 16 + s
    pltpu.sync_copy(zeros_in, zbuf)
    pltpu.sync_copy(ones_in, obuf)
    _zero_slab(zbuf, slab, s)
    plsc.subcore_barrier()

    def chunk(i, carry):
        r0 = w * ROWS_PT + i * KJ
        pltpu.sync_copy(dst2d.at[pl.ds(r0, KJ)], dbuf)
        sh = [pltpu.async_copy(obuf, slab.at[dbuf.at[j]], sem2, add=True)
              for j in range(KJ)]
        for h in sh:
            h.wait()
        return carry

    lax.fori_loop(0, NSC, chunk, 0)
    plsc.subcore_barrier()
    _flush_slab(slab, out.at[c], s)


def _make_sc_agg(npass):
    @functools.partial(
        pl.kernel, mesh=_mesh, compiler_params=_sc_params,
        out_type=jax.ShapeDtypeStruct((npass, 2, NP, 16), jnp.float32),
        scratch_types=[
            pltpu.VMEM((KJ, 128), jnp.int32),
            pltpu.VMEM((KJ, 128), jnp.int32),
            pltpu.VMEM((KJ, 128, 16), jnp.float32),
            pltpu.VMEM((ZCH, 16), jnp.float32),
            pltpu.VMEM_SHARED((NP, 16), jnp.float32),
            pltpu.SemaphoreType.DMA,
            pltpu.SemaphoreType.DMA,
        ],
    )
    def agg(*refs):
        zrefs = refs[:npass]
        src2d, dst2d, zeros_in, out = refs[npass:npass + 4]
        sbuf, dbuf, rows, zbuf, slab, sem, sem2 = refs[npass + 4:]
        c = lax.axis_index("c")
        s = lax.axis_index("s")
        w = c * 16 + s
        pltpu.sync_copy(zeros_in, zbuf)
        for p in range(npass):
            _zero_slab(zbuf, slab, s)
            plsc.subcore_barrier()

            def chunk(i, carry):
                r0 = w * ROWS_PT + i * KJ
                pltpu.sync_copy(src2d.at[pl.ds(r0, KJ)], sbuf)
                pltpu.sync_copy(dst2d.at[pl.ds(r0, KJ)], dbuf)
                gh = [
                    pltpu.async_copy(zrefs[p].at[sbuf.at[j]], rows.at[j], sem)
                    for j in range(KJ)
                ]
                sh = []
                for j in range(KJ):
                    gh[j].wait()
                    sh.append(pltpu.async_copy(
                        rows.at[j], slab.at[dbuf.at[j]], sem2, add=True))
                for h in sh:
                    h.wait()
                return carry

            lax.fori_loop(0, NSC, chunk, 0)
            plsc.subcore_barrier()
            _flush_slab(slab, out.at[p, c], s)
            if p + 1 < npass:
                plsc.subcore_barrier()

    return agg


_sc_agg1 = _make_sc_agg(1)
_sc_agg8 = _make_sc_agg(8)


def _astats_body(q0, q1, xp, dinv, w1p, b1, g1, be1, out, csum, gram):
    i = pl.program_id(0)

    @pl.when(i == 0)
    def _init():
        csum[...] = jnp.zeros_like(csum)
        gram[...] = jnp.zeros_like(gram)

    a = dinv[...] * (q0[...] + q1[...] + xp[...])
    csum[...] += jnp.sum(a, axis=0, keepdims=True)
    gram[...] += lax.dot_general(a, a, (((0,), (0,)), ((), ())),
                                 preferred_element_type=jnp.float32,
                                 precision=lax.Precision.HIGHEST)

    @pl.when(i == NB - 1)
    def _fin():
        n = jnp.float32(NN)
        cw = jnp.dot(csum[...], w1p[...],
                     preferred_element_type=jnp.float32,
                     precision=lax.Precision.HIGHEST)
        mean = cw / n + b1[...]
        t = jnp.dot(gram[...], w1p[...],
                    preferred_element_type=jnp.float32,
                    precision=lax.Precision.HIGHEST)
        d = jnp.sum(w1p[...] * t, axis=0, keepdims=True)
        e2 = d / n + 2.0 * b1[...] * cw / n + b1[...] * b1[...]
        var = e2 - mean * mean
        scale = g1[...] * lax.rsqrt(var + EPS)
        shift = be1[...] - mean * scale
        out[0:1, :] = scale
        out[1:2, :] = shift


def _aapply_body(q0, q1, xp, dinv, w1p, b1, ss, w2, *outs):
    a = dinv[...] * (q0[...] + q1[...] + xp[...])
    hp = jnp.dot(a, w1p[...], preferred_element_type=jnp.float32) + b1[...]
    h = jnp.maximum(hp * ss[0:1, :] + ss[1:2, :], 0.0)
    z = jnp.dot(h, w2[...], preferred_element_type=jnp.float32) * dinv[...]
    for p in range(8):
        outs[p][...] = z[:, 16 * p:16 * (p + 1)]


def _final_body(*refs):
    q2 = refs[0]
    zs = refs[1:9]
    dinv, bf, b2, g2, be2, out, csum, csum2, pool, cnt = refs[9:]
    i = pl.program_id(0)

    @pl.when(i == 0)
    def _init():
        csum[...] = jnp.zeros_like(csum)
        csum2[...] = jnp.zeros_like(csum2)
        pool[...] = jnp.zeros_like(pool)
        cnt[...] = jnp.zeros_like(cnt)

    q = q2[...]
    cols = [q[p, 0] + q[p, 1] + zs[p][...] for p in range(8)]
    y = dinv[...] * jnp.concatenate(cols, axis=1) + b2[...]
    csum[...] += jnp.sum(y, axis=0, keepdims=True)
    csum2[...] += jnp.sum(y * y, axis=0, keepdims=True)
    gids = lax.broadcasted_iota(jnp.int32, (BNF, GG), 1)
    oh = jnp.where(bf[...] == gids, 1.0, 0.0)
    pool[...] += lax.dot_general(oh, y, (((0,), (0,)), ((), ())),
                                 preferred_element_type=jnp.float32)
    cnt[...] += lax.dot_general(oh, jnp.ones((BNF, 1), jnp.float32),
                                (((0,), (0,)), ((), ())),
                                preferred_element_type=jnp.float32,
                                precision=lax.Precision.HIGHEST)

    @pl.when(i == NBF - 1)
    def _fin():
        n = jnp.float32(NN)
        mean = csum[...] / n
        var = csum2[...] / n - mean * mean
        scale = g2[...] * lax.rsqrt(var + EPS)
        shift = be2[...] - mean * scale
        pm = pool[...] / jnp.maximum(cnt[...], 1.0)
        pb = pm * scale + shift
        nrm = jnp.sqrt(jnp.sum(pb * pb, axis=1, keepdims=True))
        out[...] = pb / jnp.maximum(nrm, 1e-12)


def kernel(x, edge_index, batch, W1, b1, g1, be1, W2, b2, g2, be2):
    f32 = jnp.float32
    pad_e = EP - EE
    srcp = jnp.concatenate(
        [edge_index[0], jnp.full((pad_e,), NN, jnp.int32)]).reshape(-1, 128)
    dstp = jnp.concatenate(
        [edge_index[1], jnp.full((pad_e,), NN, jnp.int32)]).reshape(-1, 128)
    ones_in = jnp.ones((128, 16), f32)
    zeros_in = jnp.zeros((ZCH, 16), f32)

    degp = _sc_degree(dstp, ones_in, zeros_in)
    deg = degp[0, :NN, 0] + degp[1, :NN, 0] + 1.0
    dinv = lax.rsqrt(deg)
    dcol = dinv.reshape(NN, 1)

    xprime = jnp.zeros((NP, 16), f32).at[:NN, :9].set(x * dcol)
    q1 = _sc_agg1(xprime, srcp, dstp, zeros_in)[0]
    q10, q11 = q1[0, :NN], q1[1, :NN]
    xpn = xprime[:NN]

    w1p = jnp.zeros((16, 256), f32).at[:9].set(W1)
    row = lambda v: v.reshape(1, -1)
    nspec = lambda wdt: pl.BlockSpec((BN, wdt), lambda i: (i, 0))
    full = lambda a, b: pl.BlockSpec((a, b), lambda i: (0, 0))

    ss = pl.pallas_call(
        _astats_body,
        grid=(NB,),
        in_specs=[nspec(16), nspec(16), nspec(16), nspec(1),
                  full(16, 256), full(1, 256), full(1, 256), full(1, 256)],
        out_specs=full(2, 256),
        out_shape=jax.ShapeDtypeStruct((2, 256), f32),
        scratch_shapes=[pltpu.VMEM((1, 16), f32), pltpu.VMEM((16, 16), f32)],
    )(q10, q11, xpn, dcol, w1p, row(b1), row(g1), row(be1))

    aspec = lambda wdt: pl.BlockSpec((BNA, wdt), lambda i: (i, 0))
    zparts = pl.pallas_call(
        _aapply_body,
        grid=(NBA,),
        in_specs=[aspec(16), aspec(16), aspec(16), aspec(1),
                  full(16, 256), full(1, 256), full(2, 256), full(256, 128)],
        out_specs=[aspec(16)] * 8,
        out_shape=[jax.ShapeDtypeStruct((NP, 16), f32)] * 8,
    )(q10, q11, xpn, dcol, w1p, row(b1), ss, W2)

    q2 = _sc_agg8(*zparts, srcp, dstp, zeros_in)

    bf = batch.reshape(NN, 1)
    fspec = lambda wdt: pl.BlockSpec((BNF, wdt), lambda i: (i, 0))
    return pl.pallas_call(
        _final_body,
        grid=(NBF,),
        in_specs=[pl.BlockSpec((8, 2, BNF, 16), lambda i: (0, 0, i, 0))] +
                 [fspec(16)] * 8 +
                 [fspec(1), fspec(1),
                  full(1, 128), full(1, 128), full(1, 128)],
        out_specs=full(GG, 128),
        out_shape=jax.ShapeDtypeStruct((GG, 128), f32),
        scratch_shapes=[pltpu.VMEM((1, 128), f32), pltpu.VMEM((1, 128), f32),
                        pltpu.VMEM((GG, 128), f32), pltpu.VMEM((GG, 1), f32)],
    )(q2, *zparts, dcol, bf, row(b2), row(g2), row(be2))
